# Initial kernel scaffold; baseline (speedup 1.0000x reference)
import jax
import jax.numpy as jnp
from jax.experimental import pallas as pl


def kernel(elements_info, elements_mask, W, b, atom_embedding, type_embedding):
    raise NotImplementedError("write your pallas kernel here")


# speedup vs baseline: 2.2396x; 2.2396x over previous
import functools

import jax
import jax.numpy as jnp
from jax.experimental import pallas as pl


B, N = 1024, 50
BN = B * N
BM = 2048


def _tc_body(ff_ref, zi_ref, ti_ref, m_ref, w_ref, b_ref, az_ref, tz_ref, o_ref):
    m = m_ref[:]
    f = ff_ref[:] * m
    zf = zi_ref[:] * m
    tf = ti_ref[:] * m
    z = zf.astype(jnp.int32)
    t = tf.astype(jnp.int32)

    y = jax.lax.dot_general(f, w_ref[:], (((1,), (1,)), ((), ())),
                            preferred_element_type=jnp.float32)
    y = jnp.maximum(y + b_ref[:], 0.0)

    iota_z = jax.lax.broadcasted_iota(jnp.int32, (1, 95), 1)
    iota_t = jax.lax.broadcasted_iota(jnp.int32, (1, 6), 1)
    oh_z = (z == iota_z).astype(jnp.float32)
    oh_t = (t == iota_t).astype(jnp.float32)
    ez = jax.lax.dot_general(oh_z, az_ref[:], (((1,), (0,)), ((), ())),
                             preferred_element_type=jnp.float32)
    et = jax.lax.dot_general(oh_t, tz_ref[:], (((1,), (0,)), ((), ())),
                             preferred_element_type=jnp.float32)

    valid = m >= 0.5
    cond = valid & (z >= 1) & (z <= 94)
    pf = jnp.where(valid, y, 0.0)
    pz = jnp.where(cond, ez, 0.0)
    pt = jnp.where(cond, et, 0.0)
    out = jnp.concatenate([pf, pz, pt], axis=1) * m
    o_ref[:] = out


@jax.jit
def kernel(elements_info, elements_mask, W, b, atom_embedding, type_embedding):
    ei = elements_info.reshape(BN, 7)
    ff = ei[:, :5]
    zi = ei[:, 5:6]
    ti = ei[:, 6:7]
    m = elements_mask.reshape(BN, 1)
    b2 = b.reshape(1, 16)

    grid = (BN // BM,)
    row_spec = lambda w: pl.BlockSpec((BM, w), lambda i: (i, 0))
    full = lambda s: pl.BlockSpec(s, lambda i: (0, 0))
    out = pl.pallas_call(
        _tc_body,
        grid=grid,
        in_specs=[row_spec(5), row_spec(1), row_spec(1), row_spec(1),
                  full((16, 5)), full((1, 16)), full((95, 8)), full((6, 4))],
        out_specs=pl.BlockSpec((BM, 28), lambda i: (i, 0)),
        out_shape=jax.ShapeDtypeStruct((BN, 28), jnp.float32),
    )(ff, zi, ti, m, W, b2, atom_embedding, type_embedding)
    return out.reshape(B, N, 28)


# speedup vs baseline: 3.2067x; 1.4318x over previous
import functools

import jax
import jax.numpy as jnp
from jax import lax
from jax.experimental import pallas as pl
from jax.experimental.pallas import tpu as pltpu
from jax.experimental.pallas import tpu_sc as plsc


B, N = 1024, 50
BN = B * N
NC, NS, L = 2, 16, 16
NW = NC * NS
CHUNK = BN // NW
GROUPS = CHUNK // L

_mesh = plsc.VectorSubcoreMesh(core_axis_name="c", subcore_axis_name="s")


@functools.partial(
    pl.kernel,
    mesh=_mesh,
    out_type=jax.ShapeDtypeStruct((BN * 28,), jnp.float32),
    compiler_params=pltpu.CompilerParams(needs_layout_passes=False),
    scratch_types=[
        pltpu.VMEM((CHUNK * 7,), jnp.float32),
        pltpu.VMEM((CHUNK,), jnp.float32),
        pltpu.VMEM((CHUNK * 28,), jnp.float32),
        pltpu.VMEM((104,), jnp.float32),
        pltpu.VMEM((784,), jnp.float32),
    ],
)
def _sc_kernel(info_hbm, mask_hbm, wb_hbm, tab_hbm, out_hbm,
               in_v, m_v, o_v, wb_v, tab_v):
    wid = lax.axis_index("s") * NC + lax.axis_index("c")
    base = wid * CHUNK
    pltpu.sync_copy(info_hbm.at[pl.ds(base * 7, CHUNK * 7)], in_v)
    pltpu.sync_copy(mask_hbm.at[pl.ds(base, CHUNK)], m_v)
    pltpu.sync_copy(wb_hbm, wb_v)
    pltpu.sync_copy(tab_hbm, tab_v)

    lane = lax.iota(jnp.int32, 16)
    lane28 = lane * 28

    def _splat(ref, idx):
        return plsc.load_gather(ref, [jnp.full((L,), idx, jnp.int32)])

    def body(g, carry):
        eb = g * L
        i7 = (eb * 7) + lane * 7
        f = [plsc.load_gather(in_v, [i7 + k]) for k in range(7)]
        m = m_v[pl.ds(eb, L)]
        f = [x * m for x in f]
        z = f[5].astype(jnp.int32)
        t = f[6].astype(jnp.int32)
        valid = m >= 0.5
        cond = valid & (z >= 1) & (z <= 94)
        zc = jnp.clip(z, 0, 94)
        tc = jnp.clip(t, 0, 5)
        ob = eb * 28 + lane28
        for j in range(16):
            y = _splat(wb_v, 88 + j)
            for k in range(5):
                y = y + f[k] * _splat(wb_v, 8 + j * 5 + k)
            y = jnp.maximum(y, 0.0)
            y = jnp.where(valid, y, 0.0) * m
            plsc.store_scatter(o_v, [ob + j], y)
        zi = zc * 8
        for j in range(8):
            e = plsc.load_gather(tab_v, [zi + j])
            e = jnp.where(cond, e, 0.0) * m
            plsc.store_scatter(o_v, [ob + (16 + j)], e)
        ti = 760 + tc * 4
        for j in range(4):
            e = plsc.load_gather(tab_v, [ti + j])
            e = jnp.where(cond, e, 0.0) * m
            plsc.store_scatter(o_v, [ob + (24 + j)], e)
        return carry

    lax.fori_loop(0, GROUPS, body, 0)
    pltpu.sync_copy(o_v, out_hbm.at[pl.ds(base * 28, CHUNK * 28)])


@jax.jit
def kernel(elements_info, elements_mask, W, b, atom_embedding, type_embedding):
    info = elements_info.reshape(-1)
    mask = elements_mask.reshape(-1)
    wb = jnp.concatenate([jnp.zeros((8,), jnp.float32), W.reshape(-1), b.reshape(-1)])
    tab = jnp.concatenate([atom_embedding.reshape(-1), type_embedding.reshape(-1)])
    out = _sc_kernel(info, mask, wb, tab)
    return out.reshape(B, N, 28)


# speedup vs baseline: 9.1923x; 2.8666x over previous
import functools

import jax
import jax.numpy as jnp
from jax import lax
from jax.experimental import pallas as pl
from jax.experimental.pallas import tpu as pltpu
from jax.experimental.pallas import tpu_sc as plsc


B, N = 1024, 50
NC, NS, L = 2, 16, 16
NT_N = 7
NB = 4
BQ = B // NB
GROUPS = 8 * BQ // L

_mesh = plsc.VectorSubcoreMesh(core_axis_name="c", subcore_axis_name="s")


@functools.partial(
    pl.kernel,
    mesh=_mesh,
    out_type=jax.ShapeDtypeStruct((28, N, B), jnp.float32),
    compiler_params=pltpu.CompilerParams(needs_layout_passes=False),
    scratch_types=[
        pltpu.VMEM((7, 8, BQ), jnp.float32),
        pltpu.VMEM((8, BQ), jnp.float32),
        pltpu.VMEM((28, 8, BQ), jnp.float32),
        pltpu.VMEM((104,), jnp.float32),
        pltpu.VMEM((784,), jnp.float32),
    ],
)
def _sc_kernel(info_hbm, mask_hbm, wb_hbm, tab_hbm, out_hbm,
               in_v, m_v, o_v, wb_v, tab_v):
    wid = lax.axis_index("s") * NC + lax.axis_index("c")

    @pl.when(wid < NT_N * NB)
    def _():
        tn = wid // NB
        n0 = tn * 8
        b0 = (wid % NB) * BQ
        pltpu.sync_copy(wb_hbm, wb_v)
        pltpu.sync_copy(tab_hbm, tab_v)
        pltpu.sync_copy(info_hbm.at[:, pl.ds(n0, 8), pl.ds(b0, BQ)], in_v)
        pltpu.sync_copy(mask_hbm.at[pl.ds(n0, 8), pl.ds(b0, BQ)], m_v)

        def _splat(ref, idx):
            return plsc.load_gather(ref, [jnp.full((L,), idx, jnp.int32)])

        wvec = [[_splat(wb_v, 8 + j * 5 + k) for k in range(5)]
                for j in range(16)]
        bvec = [_splat(wb_v, 88 + j) for j in range(16)]

        def body(g, carry):
            r = g // (BQ // L)
            c = (g % (BQ // L)) * L
            m = m_v[r, pl.ds(c, L)]
            f = [in_v[k, r, pl.ds(c, L)] * m for k in range(7)]
            z = f[5].astype(jnp.int32)
            t = f[6].astype(jnp.int32)
            valid = m >= 0.5
            cond = valid & (z >= 1) & (z <= 94)
            zc = jnp.clip(z, 0, 94)
            tc = jnp.clip(t, 0, 5)
            for j in range(16):
                y = bvec[j]
                for k in range(5):
                    y = y + f[k] * wvec[j][k]
                y = jnp.maximum(y, 0.0)
                y = jnp.where(valid, y, 0.0) * m
                o_v[j, r, pl.ds(c, L)] = y
            zi = zc * 8
            for j in range(8):
                e = plsc.load_gather(tab_v, [zi + j])
                e = jnp.where(cond, e, 0.0) * m
                o_v[16 + j, r, pl.ds(c, L)] = e
            ti = 760 + tc * 4
            for j in range(4):
                e = plsc.load_gather(tab_v, [ti + j])
                e = jnp.where(cond, e, 0.0) * m
                o_v[24 + j, r, pl.ds(c, L)] = e
            return carry

        lax.fori_loop(0, GROUPS, body, 0)
        pltpu.sync_copy(o_v, out_hbm.at[:, pl.ds(n0, 8), pl.ds(b0, BQ)])


@jax.jit
def kernel(elements_info, elements_mask, W, b, atom_embedding, type_embedding):
    eiT = jnp.transpose(elements_info, (2, 1, 0))
    mT = elements_mask.T
    wb = jnp.concatenate([jnp.zeros((8,), jnp.float32), W.reshape(-1),
                          b.reshape(-1)])
    tab = jnp.concatenate([atom_embedding.reshape(-1),
                           type_embedding.reshape(-1)])
    outT = _sc_kernel(eiT, mT, wb, tab)
    return jnp.transpose(outT, (2, 1, 0))


# speedup vs baseline: 10.4863x; 1.1408x over previous
import functools

import jax
import jax.numpy as jnp
from jax import lax
from jax.experimental import pallas as pl
from jax.experimental.pallas import tpu as pltpu
from jax.experimental.pallas import tpu_sc as plsc


B, N = 1024, 50
NC, NS, L = 2, 16, 16
NT_N = 7
NB = 4
BQ = B // NB
GROUPS = 8 * BQ // L

_mesh = plsc.VectorSubcoreMesh(core_axis_name="c", subcore_axis_name="s")


@functools.partial(
    pl.kernel,
    mesh=_mesh,
    out_type=jax.ShapeDtypeStruct((28, N, B), jnp.float32),
    compiler_params=pltpu.CompilerParams(needs_layout_passes=False),
    scratch_types=[
        pltpu.VMEM((7, 8, BQ), jnp.float32),
        pltpu.VMEM((8, BQ), jnp.float32),
        pltpu.VMEM((28, 8, BQ), jnp.float32),
        pltpu.VMEM((104,), jnp.float32),
        pltpu.VMEM((784,), jnp.float32),
    ],
)
def _sc_kernel(info_hbm, mask_hbm, wb_hbm, tab_hbm, out_hbm,
               in_v, m_v, o_v, wb_v, tab_v):
    wid = lax.axis_index("s") * NC + lax.axis_index("c")

    @pl.when(wid < NT_N * NB)
    def _():
        tn = wid // NB
        n0 = tn * 8
        b0 = (wid % NB) * BQ
        pltpu.sync_copy(wb_hbm, wb_v)
        pltpu.sync_copy(tab_hbm, tab_v)
        pltpu.sync_copy(info_hbm.at[:, pl.ds(n0, 8), pl.ds(b0, BQ)], in_v)
        pltpu.sync_copy(mask_hbm.at[pl.ds(n0, 8), pl.ds(b0, BQ)], m_v)

        def _splat(ref, idx):
            return plsc.load_gather(ref, [jnp.full((L,), idx, jnp.int32)])

        wvec = [[_splat(wb_v, 8 + j * 5 + k) for k in range(5)]
                for j in range(16)]
        bvec = [_splat(wb_v, 88 + j) for j in range(16)]

        zero = jnp.zeros((L,), jnp.float32)

        @plsc.parallel_loop(0, GROUPS, unroll=4)
        def _body(g):
            r = g // (BQ // L)
            c = (g % (BQ // L)) * L
            m = m_v[r, pl.ds(c, L)]
            f = [in_v[k, r, pl.ds(c, L)] * m for k in range(7)]
            z = f[5].astype(jnp.int32)
            t = f[6].astype(jnp.int32)
            valid = m >= 0.5
            cond = valid & (z >= 1) & (z <= 94)
            mf = jnp.where(valid, m, zero)
            cf = jnp.where(cond, m, zero)
            zc = jnp.clip(z, 0, 94)
            tc = jnp.clip(t, 0, 5)
            for j in range(16):
                y = bvec[j]
                for k in range(5):
                    y = y + f[k] * wvec[j][k]
                o_v[j, r, pl.ds(c, L)] = jnp.maximum(y, 0.0) * mf
            zi = zc * 8
            for j in range(8):
                e = plsc.load_gather(tab_v, [zi + j])
                o_v[16 + j, r, pl.ds(c, L)] = e * cf
            ti = 760 + tc * 4
            for j in range(4):
                e = plsc.load_gather(tab_v, [ti + j])
                o_v[24 + j, r, pl.ds(c, L)] = e * cf
        pltpu.sync_copy(o_v, out_hbm.at[:, pl.ds(n0, 8), pl.ds(b0, BQ)])


@jax.jit
def kernel(elements_info, elements_mask, W, b, atom_embedding, type_embedding):
    eiT = jnp.transpose(elements_info, (2, 1, 0))
    mT = elements_mask.T
    wb = jnp.concatenate([jnp.zeros((8,), jnp.float32), W.reshape(-1),
                          b.reshape(-1)])
    tab = jnp.concatenate([atom_embedding.reshape(-1),
                           type_embedding.reshape(-1)])
    outT = _sc_kernel(eiT, mT, wb, tab)
    return jnp.transpose(outT, (2, 1, 0))


# speedup vs baseline: 11.7034x; 1.1161x over previous
import functools

import jax
import jax.numpy as jnp
from jax import lax
from jax.experimental import pallas as pl
from jax.experimental.pallas import tpu as pltpu
from jax.experimental.pallas import tpu_sc as plsc


B, N = 1024, 50
NC, NS, L = 2, 16, 16
NT_N = 7
NB = 4
BQ = B // NB
GROUPS = 8 * BQ // L

_mesh = plsc.VectorSubcoreMesh(core_axis_name="c", subcore_axis_name="s")


@functools.partial(
    pl.kernel,
    mesh=_mesh,
    out_type=jax.ShapeDtypeStruct((28, N, B), jnp.float32),
    compiler_params=pltpu.CompilerParams(needs_layout_passes=False),
    scratch_types=[
        pltpu.VMEM((7, 8, BQ), jnp.float32),
        pltpu.VMEM((8, BQ), jnp.float32),
        pltpu.VMEM((28, 8, BQ), jnp.float32),
        pltpu.VMEM((104,), jnp.float32),
        pltpu.VMEM((784,), jnp.float32),
    ],
)
def _sc_kernel(info_hbm, mask_hbm, wb_hbm, tab_hbm, out_hbm,
               in_v, m_v, o_v, wb_v, tab_v):
    wid = lax.axis_index("s") * NC + lax.axis_index("c")

    @pl.when(wid < NT_N * NB)
    def _():
        tn = wid // NB
        n0 = tn * 8
        b0 = (wid % NB) * BQ
        pltpu.sync_copy(wb_hbm, wb_v)
        pltpu.sync_copy(tab_hbm, tab_v)
        pltpu.sync_copy(info_hbm.at[:, pl.ds(n0, 8), pl.ds(b0, BQ)], in_v)
        pltpu.sync_copy(mask_hbm.at[pl.ds(n0, 8), pl.ds(b0, BQ)], m_v)

        def _splat(ref, idx):
            return plsc.load_gather(ref, [jnp.full((L,), idx, jnp.int32)])

        zero = jnp.zeros((L,), jnp.float32)
        JB = 4

        for jb in range(16 // JB):
            wv = [[_splat(wb_v, 8 + (jb * JB + jj) * 5 + k) for k in range(5)]
                  for jj in range(JB)]
            bv = [_splat(wb_v, 88 + jb * JB + jj) for jj in range(JB)]

            @plsc.parallel_loop(0, GROUPS, unroll=2)
            def _lin(g):
                r = g // (BQ // L)
                c = (g % (BQ // L)) * L
                m = m_v[r, pl.ds(c, L)]
                f = [in_v[k, r, pl.ds(c, L)] for k in range(5)]
                mf = jnp.where(m >= 0.5, m, zero)
                for jj in range(JB):
                    s = f[0] * wv[jj][0]
                    for k in range(1, 5):
                        s = s + f[k] * wv[jj][k]
                    y = s * m + bv[jj]
                    o_v[jb * JB + jj, r, pl.ds(c, L)] = (
                        jnp.maximum(y, 0.0) * mf)

        @plsc.parallel_loop(0, GROUPS, unroll=2)
        def _emb(g):
            r = g // (BQ // L)
            c = (g % (BQ // L)) * L
            m = m_v[r, pl.ds(c, L)]
            z = (in_v[5, r, pl.ds(c, L)] * m).astype(jnp.int32)
            t = (in_v[6, r, pl.ds(c, L)] * m).astype(jnp.int32)
            cond = (m >= 0.5) & (z >= 1) & (z <= 94)
            cf = jnp.where(cond, m, zero)
            zi = jnp.clip(z, 0, 94) * 8
            for j in range(8):
                e = plsc.load_gather(tab_v, [zi + j])
                o_v[16 + j, r, pl.ds(c, L)] = e * cf
            ti = 760 + jnp.clip(t, 0, 5) * 4
            for j in range(4):
                e = plsc.load_gather(tab_v, [ti + j])
                o_v[24 + j, r, pl.ds(c, L)] = e * cf
        pltpu.sync_copy(o_v, out_hbm.at[:, pl.ds(n0, 8), pl.ds(b0, BQ)])


@jax.jit
def kernel(elements_info, elements_mask, W, b, atom_embedding, type_embedding):
    eiT = jnp.transpose(elements_info, (2, 1, 0))
    mT = elements_mask.T
    wb = jnp.concatenate([jnp.zeros((8,), jnp.float32), W.reshape(-1),
                          b.reshape(-1)])
    tab = jnp.concatenate([atom_embedding.reshape(-1),
                           type_embedding.reshape(-1)])
    outT = _sc_kernel(eiT, mT, wb, tab)
    return jnp.transpose(outT, (2, 1, 0))
